# Initial kernel scaffold; baseline (speedup 1.0000x reference)
import jax
import jax.numpy as jnp
from jax.experimental import pallas as pl


def kernel(seq_pos, seq_neg, adj, diff, sparse, msk, samp_bias1, samp_bias2, W_adj1, b_adj1, a_adj1, W_diff1, b_diff1, a_diff1, W_adj2, b_adj2, a_adj2, W_diff2, b_diff2, a_diff2):
    raise NotImplementedError("write your pallas kernel here")


# speedup vs baseline: 1.6107x; 1.6107x over previous
import functools

import jax
import jax.numpy as jnp
from jax.experimental import pallas as pl


def _mm_kernel(x_ref, w_ref, o_ref):
    o_ref[...] = jnp.dot(x_ref[...], w_ref[...],
                         preferred_element_type=jnp.float32)


def _matmul(x, w, bl):
    n, k = x.shape
    m = w.shape[1]
    return pl.pallas_call(
        _mm_kernel,
        grid=(n // bl,),
        in_specs=[
            pl.BlockSpec((bl, k), lambda i: (i, 0)),
            pl.BlockSpec((k, m), lambda i: (0, 0)),
        ],
        out_specs=pl.BlockSpec((bl, m), lambda i: (i, 0)),
        out_shape=jax.ShapeDtypeStruct((n, m), jnp.float32),
    )(x, w)


def _gcn_kernel(a_ref, f_ref, b_ref, al_ref, o_ref):
    acc = jnp.dot(a_ref[...], f_ref[...], preferred_element_type=jnp.float32)
    x = acc + b_ref[...]
    o_ref[...] = jnp.where(x >= 0, x, x * al_ref[...])


def _gcn(a, f, b_row, al_row, bm):
    n = a.shape[0]
    c = f.shape[1]
    return pl.pallas_call(
        _gcn_kernel,
        grid=(n // bm,),
        in_specs=[
            pl.BlockSpec((bm, n), lambda i: (i, 0)),
            pl.BlockSpec((n, c), lambda i: (0, 0)),
            pl.BlockSpec((1, c), lambda i: (0, 0)),
            pl.BlockSpec((1, c), lambda i: (0, 0)),
        ],
        out_specs=pl.BlockSpec((bm, c), lambda i: (i, 0)),
        out_shape=jax.ShapeDtypeStruct((n, c), jnp.float32),
    )(a, f, b_row, al_row)


def _blockdiag2(wt):
    k, h = wt.shape
    z = jnp.zeros((k, h), jnp.float32)
    top = jnp.concatenate([wt, z], axis=1)
    bot = jnp.concatenate([z, wt], axis=1)
    return jnp.concatenate([top, bot], axis=0)


def _pair(b, a):
    b128 = jnp.concatenate([b, b])[None, :]
    a128 = jnp.broadcast_to(a.reshape(1, 1), (1, 2 * b.shape[0]))
    return b128, a128


def kernel(seq_pos, seq_neg, adj, diff, sparse, msk, samp_bias1, samp_bias2,
           W_adj1, b_adj1, a_adj1, W_diff1, b_diff1, a_diff1,
           W_adj2, b_adj2, a_adj2, W_diff2, b_diff2, a_diff2):
    n = seq_pos.shape[1]
    bm = 400 if n % 400 == 0 else n
    bl = 2000 if n % 2000 == 0 else n

    a2 = adj.reshape(n, n)
    d2 = diff.reshape(n, n)
    x = jnp.concatenate([seq_pos[0], seq_neg[0]], axis=1)

    f_adj1 = _matmul(x, _blockdiag2(W_adj1.T), bl)
    f_diff1 = _matmul(x, _blockdiag2(W_diff1.T), bl)

    b_a1, al_a1 = _pair(b_adj1, a_adj1)
    b_d1, al_d1 = _pair(b_diff1, a_diff1)
    h_adj1 = _gcn(a2, f_adj1, b_a1, al_a1, bm)
    h_diff1 = _gcn(d2, f_diff1, b_d1, al_d1, bm)

    f_adj2 = _matmul(h_adj1, _blockdiag2(W_adj2.T), bl)
    f_diff2 = _matmul(h_diff1, _blockdiag2(W_diff2.T), bl)

    b_a2, al_a2 = _pair(b_adj2, a_adj2)
    b_d2, al_d2 = _pair(b_diff2, a_diff2)
    h_adj2 = _gcn(a2, f_adj2, b_a2, al_a2, bm)
    h_diff2 = _gcn(d2, f_diff2, b_d2, al_d2, bm)

    dh = W_adj1.shape[0]
    def split(h):
        return h[None, :, :dh], h[None, :, dh:]

    h_adj_l1_pos, h_adj_l1_neg = split(h_adj1)
    h_diff_l1_pos, h_diff_l1_neg = split(h_diff1)
    h_adj_l2_pos, h_adj_l2_neg = split(h_adj2)
    h_diff_l2_pos, h_diff_l2_neg = split(h_diff2)

    return (h_adj_l1_pos, h_diff_l1_pos, h_adj_l2_pos, h_diff_l2_pos,
            h_adj_l1_neg, h_diff_l1_neg, h_adj_l2_neg, h_diff_l2_neg)


# speedup vs baseline: 1.6291x; 1.0114x over previous
import functools

import jax
import jax.numpy as jnp
from jax.experimental import pallas as pl


def _mm_kernel(x_ref, w_ref, o_ref):
    o_ref[...] = jnp.dot(x_ref[...], w_ref[...],
                         preferred_element_type=jnp.float32).astype(o_ref.dtype)


def _matmul(x, w, bl, out_dtype=jnp.float32):
    n, k = x.shape
    m = w.shape[1]
    return pl.pallas_call(
        _mm_kernel,
        grid=(n // bl,),
        in_specs=[
            pl.BlockSpec((bl, k), lambda i: (i, 0)),
            pl.BlockSpec((k, m), lambda i: (0, 0)),
        ],
        out_specs=pl.BlockSpec((bl, m), lambda i: (i, 0)),
        out_shape=jax.ShapeDtypeStruct((n, m), out_dtype),
    )(x, w)


def _gcn_kernel(a_ref, f_ref, b_ref, al_ref, o_ref):
    a_bf = a_ref[...].astype(jnp.bfloat16)
    acc = jnp.dot(a_bf, f_ref[...], preferred_element_type=jnp.float32)
    x = acc + b_ref[...]
    o_ref[...] = jnp.where(x >= 0, x, x * al_ref[...])


def _gcn(a, f, b_row, al_row, bm):
    n = a.shape[0]
    c = f.shape[1]
    return pl.pallas_call(
        _gcn_kernel,
        grid=(n // bm,),
        in_specs=[
            pl.BlockSpec((bm, n), lambda i: (i, 0)),
            pl.BlockSpec((n, c), lambda i: (0, 0)),
            pl.BlockSpec((1, c), lambda i: (0, 0)),
            pl.BlockSpec((1, c), lambda i: (0, 0)),
        ],
        out_specs=pl.BlockSpec((bm, c), lambda i: (i, 0)),
        out_shape=jax.ShapeDtypeStruct((n, c), jnp.float32),
    )(a, f, b_row, al_row)


def _blockdiag2(wt):
    k, h = wt.shape
    z = jnp.zeros((k, h), jnp.float32)
    top = jnp.concatenate([wt, z], axis=1)
    bot = jnp.concatenate([z, wt], axis=1)
    return jnp.concatenate([top, bot], axis=0)


def _pair(b, a):
    b128 = jnp.concatenate([b, b])[None, :]
    a128 = jnp.broadcast_to(a.reshape(1, 1), (1, 2 * b.shape[0]))
    return b128, a128


def kernel(seq_pos, seq_neg, adj, diff, sparse, msk, samp_bias1, samp_bias2,
           W_adj1, b_adj1, a_adj1, W_diff1, b_diff1, a_diff1,
           W_adj2, b_adj2, a_adj2, W_diff2, b_diff2, a_diff2):
    n = seq_pos.shape[1]
    bm = 400 if n % 400 == 0 else n
    bl = 2000 if n % 2000 == 0 else n

    a2 = adj.reshape(n, n)
    d2 = diff.reshape(n, n)
    x = jnp.concatenate([seq_pos[0], seq_neg[0]], axis=1)

    f_adj1 = _matmul(x, _blockdiag2(W_adj1.T), bl, jnp.bfloat16)
    f_diff1 = _matmul(x, _blockdiag2(W_diff1.T), bl, jnp.bfloat16)

    b_a1, al_a1 = _pair(b_adj1, a_adj1)
    b_d1, al_d1 = _pair(b_diff1, a_diff1)
    h_adj1 = _gcn(a2, f_adj1, b_a1, al_a1, bm)
    h_diff1 = _gcn(d2, f_diff1, b_d1, al_d1, bm)

    f_adj2 = _matmul(h_adj1, _blockdiag2(W_adj2.T), bl, jnp.bfloat16)
    f_diff2 = _matmul(h_diff1, _blockdiag2(W_diff2.T), bl, jnp.bfloat16)

    b_a2, al_a2 = _pair(b_adj2, a_adj2)
    b_d2, al_d2 = _pair(b_diff2, a_diff2)
    h_adj2 = _gcn(a2, f_adj2, b_a2, al_a2, bm)
    h_diff2 = _gcn(d2, f_diff2, b_d2, al_d2, bm)

    dh = W_adj1.shape[0]
    def split(h):
        return h[None, :, :dh], h[None, :, dh:]

    h_adj_l1_pos, h_adj_l1_neg = split(h_adj1)
    h_diff_l1_pos, h_diff_l1_neg = split(h_diff1)
    h_adj_l2_pos, h_adj_l2_neg = split(h_adj2)
    h_diff_l2_pos, h_diff_l2_neg = split(h_diff2)

    return (h_adj_l1_pos, h_diff_l1_pos, h_adj_l2_pos, h_diff_l2_pos,
            h_adj_l1_neg, h_diff_l1_neg, h_adj_l2_neg, h_diff_l2_neg)


# speedup vs baseline: 1.7642x; 1.0829x over previous
import jax
import jax.numpy as jnp
from jax.experimental import pallas as pl

_BF = jnp.bfloat16


def _lin1_kernel(xp_ref, xn_ref, wa_ref, wd_ref, oa_ref, od_ref):
    xp = xp_ref[...].astype(_BF)
    xn = xn_ref[...].astype(_BF)

    def lin(w):
        p = jnp.dot(xp, w, preferred_element_type=jnp.float32)
        q = jnp.dot(xn, w, preferred_element_type=jnp.float32)
        return jnp.concatenate([p, q], axis=1).astype(_BF)

    oa_ref[...] = lin(wa_ref[...])
    od_ref[...] = lin(wd_ref[...])


def _gcn1_kernel(a_ref, f_ref, b_ref, al_ref, w2_ref, op_ref, on_ref, f2_ref):
    a_bf = a_ref[...].astype(_BF)
    h = jnp.dot(a_bf, f_ref[...], preferred_element_type=jnp.float32)
    h = h + b_ref[...]
    h = jnp.where(h >= 0, h, h * al_ref[...])
    dh = op_ref.shape[1]
    op_ref[...] = h[:, :dh]
    on_ref[...] = h[:, dh:]
    f2_ref[...] = jnp.dot(h.astype(_BF), w2_ref[...],
                          preferred_element_type=jnp.float32).astype(_BF)


def _gcn2_kernel(a_ref, f_ref, b_ref, al_ref, op_ref, on_ref):
    a_bf = a_ref[...].astype(_BF)
    h = jnp.dot(a_bf, f_ref[...], preferred_element_type=jnp.float32)
    h = h + b_ref[...]
    h = jnp.where(h >= 0, h, h * al_ref[...])
    dh = op_ref.shape[1]
    op_ref[...] = h[:, :dh]
    on_ref[...] = h[:, dh:]


def _blockdiag2(wt):
    k, h = wt.shape
    z = jnp.zeros((k, h), wt.dtype)
    top = jnp.concatenate([wt, z], axis=1)
    bot = jnp.concatenate([z, wt], axis=1)
    return jnp.concatenate([top, bot], axis=0)


def _pair(b, a):
    b128 = jnp.concatenate([b, b])[None, :]
    a128 = jnp.broadcast_to(a.reshape(1, 1), (1, 2 * b.shape[0]))
    return b128, a128


def _lin1(xp, xn, wa_t, wd_t, bl):
    n, k = xp.shape
    dh = wa_t.shape[1]
    out = jax.ShapeDtypeStruct((n, 2 * dh), _BF)
    return pl.pallas_call(
        _lin1_kernel,
        grid=(n // bl,),
        in_specs=[
            pl.BlockSpec((bl, k), lambda i: (i, 0)),
            pl.BlockSpec((bl, k), lambda i: (i, 0)),
            pl.BlockSpec((k, dh), lambda i: (0, 0)),
            pl.BlockSpec((k, dh), lambda i: (0, 0)),
        ],
        out_specs=[pl.BlockSpec((bl, 2 * dh), lambda i: (i, 0))] * 2,
        out_shape=[out, out],
    )(xp, xn, wa_t, wd_t)


def _gcn1(a, f, b_row, al_row, w2big, bm):
    n = a.shape[0]
    c = f.shape[1]
    dh = c // 2
    return pl.pallas_call(
        _gcn1_kernel,
        grid=(n // bm,),
        in_specs=[
            pl.BlockSpec((bm, n), lambda i: (i, 0)),
            pl.BlockSpec((n, c), lambda i: (0, 0)),
            pl.BlockSpec((1, c), lambda i: (0, 0)),
            pl.BlockSpec((1, c), lambda i: (0, 0)),
            pl.BlockSpec((c, c), lambda i: (0, 0)),
        ],
        out_specs=[
            pl.BlockSpec((bm, dh), lambda i: (i, 0)),
            pl.BlockSpec((bm, dh), lambda i: (i, 0)),
            pl.BlockSpec((bm, c), lambda i: (i, 0)),
        ],
        out_shape=[
            jax.ShapeDtypeStruct((n, dh), jnp.float32),
            jax.ShapeDtypeStruct((n, dh), jnp.float32),
            jax.ShapeDtypeStruct((n, c), _BF),
        ],
    )(a, f, b_row, al_row, w2big)


def _gcn2(a, f, b_row, al_row, bm):
    n = a.shape[0]
    c = f.shape[1]
    dh = c // 2
    return pl.pallas_call(
        _gcn2_kernel,
        grid=(n // bm,),
        in_specs=[
            pl.BlockSpec((bm, n), lambda i: (i, 0)),
            pl.BlockSpec((n, c), lambda i: (0, 0)),
            pl.BlockSpec((1, c), lambda i: (0, 0)),
            pl.BlockSpec((1, c), lambda i: (0, 0)),
        ],
        out_specs=[
            pl.BlockSpec((bm, dh), lambda i: (i, 0)),
            pl.BlockSpec((bm, dh), lambda i: (i, 0)),
        ],
        out_shape=[
            jax.ShapeDtypeStruct((n, dh), jnp.float32),
            jax.ShapeDtypeStruct((n, dh), jnp.float32),
        ],
    )(a, f, b_row, al_row)


def kernel(seq_pos, seq_neg, adj, diff, sparse, msk, samp_bias1, samp_bias2,
           W_adj1, b_adj1, a_adj1, W_diff1, b_diff1, a_diff1,
           W_adj2, b_adj2, a_adj2, W_diff2, b_diff2, a_diff2):
    n = seq_pos.shape[1]
    bm = 400 if n % 400 == 0 else n
    bl = 2000 if n % 2000 == 0 else n

    a2 = adj.reshape(n, n)
    d2 = diff.reshape(n, n)
    xp = seq_pos.reshape(n, -1)
    xn = seq_neg.reshape(n, -1)

    f_adj1, f_diff1 = _lin1(xp, xn, W_adj1.T.astype(_BF),
                            W_diff1.T.astype(_BF), bl)

    w2a = _blockdiag2(W_adj2.T.astype(_BF))
    w2d = _blockdiag2(W_diff2.T.astype(_BF))

    b_a1, al_a1 = _pair(b_adj1, a_adj1)
    b_d1, al_d1 = _pair(b_diff1, a_diff1)
    ha1p, ha1n, f_adj2 = _gcn1(a2, f_adj1, b_a1, al_a1, w2a, bm)
    hd1p, hd1n, f_diff2 = _gcn1(d2, f_diff1, b_d1, al_d1, w2d, bm)

    b_a2, al_a2 = _pair(b_adj2, a_adj2)
    b_d2, al_d2 = _pair(b_diff2, a_diff2)
    ha2p, ha2n = _gcn2(a2, f_adj2, b_a2, al_a2, bm)
    hd2p, hd2n = _gcn2(d2, f_diff2, b_d2, al_d2, bm)

    def lift(h):
        return h[None]

    return (lift(ha1p), lift(hd1p), lift(ha2p), lift(hd2p),
            lift(ha1n), lift(hd1n), lift(ha2n), lift(hd2n))
